# Initial kernel scaffold; baseline (speedup 1.0000x reference)
import jax
import jax.numpy as jnp
from jax.experimental import pallas as pl


def kernel(x, w1, w2, wg):
    raise NotImplementedError("write your pallas kernel here")


# speedup vs baseline: 2.0556x; 2.0556x over previous
import jax
import jax.numpy as jnp
from jax.experimental import pallas as pl
from jax.experimental.pallas import tpu as pltpu

HIDDEN = 2048
BLOCK = 1024
NB = 8
NACT = 7
TM = 512


def _mlp_body(mask_ref, x_ref, w1_ref, w2_ref, o_ref):
    i = pl.program_id(1)
    h = jnp.dot(x_ref[...], w1_ref[...], preferred_element_type=jnp.float32)
    h = 0.5 * h * (1.0 + jax.lax.erf(h * jnp.float32(0.7071067811865476)))
    y = jnp.dot(h, w2_ref[...], preferred_element_type=jnp.float32)
    onehot = (jax.lax.broadcasted_iota(jnp.int32, (1, 8), 1) == i).astype(jnp.float32)
    mcol = jnp.sum(mask_ref[...] * onehot, axis=1, keepdims=True)

    @pl.when(i == 0)
    def _():
        o_ref[...] = jnp.zeros_like(o_ref)

    o_ref[...] += y * mcol


def kernel(x, w1, w2, wg):
    orig_shape = x.shape
    gate_logits = x @ wg
    gidx = jnp.argmax(gate_logits, axis=-1).reshape(-1)
    xf = x.reshape(-1, HIDDEN)
    T = xf.shape[0]
    H = jnp.max(gidx)
    iidx = jnp.arange(1, NB + 1)
    mask = ((gidx[:, None] >= iidx[None, :]) & (iidx[None, :] < H)).astype(jnp.float32)

    nt = T // TM
    out = pl.pallas_call(
        _mlp_body,
        grid=(nt, NACT),
        in_specs=[
            pl.BlockSpec((TM, NB), lambda j, i: (j, 0)),
            pl.BlockSpec((TM, HIDDEN), lambda j, i: (j, 0)),
            pl.BlockSpec((HIDDEN, BLOCK), lambda j, i: (0, i)),
            pl.BlockSpec((BLOCK, HIDDEN), lambda j, i: (i, 0)),
        ],
        out_specs=pl.BlockSpec((TM, HIDDEN), lambda j, i: (j, 0)),
        out_shape=jax.ShapeDtypeStruct((T, HIDDEN), jnp.float32),
    )(mask, xf, w1, w2)
    return (out.reshape(orig_shape), gate_logits)
